# baseline (speedup 1.0000x reference)
import functools

import jax
import jax.numpy as jnp
from jax import lax
from jax.experimental import pallas as pl
from jax.experimental.pallas import tpu as pltpu
from jax.experimental.pallas import tpu_sc as plsc

_NC = 2
_NS = 16
_NW = _NC * _NS
_DIM = 64
_CHUNK = 128


@functools.cache
def _build_gather(n_rows: int):
    per_w = n_rows // _NW
    n_chunks = per_w // _CHUNK
    mesh = plsc.VectorSubcoreMesh(
        core_axis_name="c", subcore_axis_name="s",
        num_cores=_NC, num_subcores=_NS,
    )

    @functools.partial(
        pl.kernel,
        out_type=jax.ShapeDtypeStruct((n_rows, _DIM), jnp.float32),
        mesh=mesh,
        scratch_types=[
            pltpu.VMEM((_CHUNK,), jnp.int32),
            pltpu.VMEM((_CHUNK, _DIM), jnp.float32),
            pltpu.SemaphoreType.DMA,
        ],
        compiler_params=pltpu.CompilerParams(use_tc_tiling_on_sc=False),
    )
    def gather(idx_hbm, table_hbm, out_hbm, idx_v, rows_v, sem):
        wid = lax.axis_index("s") * _NC + lax.axis_index("c")
        base = wid * per_w

        def chunk(g, carry):
            off = base + g * _CHUNK
            pltpu.sync_copy(idx_hbm.at[pl.ds(off, _CHUNK)], idx_v)
            pltpu.async_copy(table_hbm.at[idx_v], rows_v, sem).wait()
            pltpu.sync_copy(rows_v, out_hbm.at[pl.ds(off, _CHUNK)])
            return carry

        lax.fori_loop(0, n_chunks, chunk, 0)

    return gather


def kernel(item_seq, item_emb_weight):
    b, l = item_seq.shape
    n = b * l
    idx = item_seq.reshape(n).astype(jnp.int32)
    out = _build_gather(n)(idx, item_emb_weight)
    return out.reshape(b, l, _DIM)


# speedup vs baseline: 1.1909x; 1.1909x over previous
import functools

import jax
import jax.numpy as jnp
from jax import lax
from jax.experimental import pallas as pl
from jax.experimental.pallas import tpu as pltpu
from jax.experimental.pallas import tpu_sc as plsc

_NC = 2
_NS = 16
_NW = _NC * _NS
_DIM = 64
_CHUNK = 128
_NBUF = 4
_FD = 2


@functools.cache
def _build_gather(n_rows: int):
    per_w = n_rows // _NW
    n_chunks = per_w // _CHUNK
    n_super = n_chunks // _NBUF
    mesh = plsc.VectorSubcoreMesh(
        core_axis_name="c", subcore_axis_name="s",
        num_cores=_NC, num_subcores=_NS,
    )

    @functools.partial(
        pl.kernel,
        out_type=jax.ShapeDtypeStruct((n_rows, _DIM), jnp.float32),
        mesh=mesh,
        scratch_types=[
            pltpu.VMEM((n_chunks, _CHUNK), jnp.int32),
            pltpu.VMEM((_NBUF, _CHUNK, _DIM), jnp.float32),
        ]
        + [pltpu.SemaphoreType.DMA] * (2 * _NBUF),
        compiler_params=pltpu.CompilerParams(use_tc_tiling_on_sc=False),
    )
    def gather(idx_hbm, table_hbm, out_hbm, idx_v, rows_v, *sems):
        gs = sems[:_NBUF]
        os_ = sems[_NBUF:]
        wid = lax.axis_index("s") * _NC + lax.axis_index("c")
        base = wid * per_w

        pltpu.sync_copy(idx_hbm.at[wid], idx_v)

        def fire(c, b):
            pltpu.async_copy(table_hbm.at[idx_v.at[c]], rows_v.at[b], gs[b])

        def drain_gather(c, b):
            pltpu.make_async_copy(
                table_hbm.at[idx_v.at[c]], rows_v.at[b], gs[b]).wait()

        def out_start(c, b):
            pltpu.async_copy(
                rows_v.at[b], out_hbm.at[pl.ds(base + c * _CHUNK, _CHUNK)],
                os_[b])

        def out_wait(b):
            pltpu.make_async_copy(
                rows_v.at[b], out_hbm.at[pl.ds(base, _CHUNK)], os_[b]).wait()

        for c0 in range(_FD):
            fire(c0, c0)

        def super_iter(s, carry):
            for b in range(_NBUF):
                g = s * _NBUF + b
                bw = (b + _FD) % _NBUF
                if b + _FD < _NBUF:
                    @pl.when(s >= 1)
                    def _():
                        out_wait(bw)
                        fire(g + _FD, bw)
                    @pl.when(s == 0)
                    def _():
                        fire(g + _FD, bw)
                else:
                    out_wait(bw)
                    @pl.when(s < n_super - 1)
                    def _():
                        fire(g + _FD, bw)
                drain_gather(g, b)
                out_start(g, b)
            return carry

        lax.fori_loop(0, n_super, super_iter, 0)

        for j in range(_NBUF - _FD):
            out_wait((_FD + j) % _NBUF)

    return gather


def kernel(item_seq, item_emb_weight):
    b, l = item_seq.shape
    n = b * l
    per_w = n // _NW
    idx = item_seq.reshape(_NW, per_w // _CHUNK, _CHUNK).astype(jnp.int32)
    out = _build_gather(n)(idx, item_emb_weight)
    return out.reshape(b, l, _DIM)
